# baseline (speedup 1.0000x reference)
import functools

import jax
import jax.numpy as jnp
from jax import lax
from jax.experimental import pallas as pl
from jax.experimental.pallas import tpu as pltpu
from jax.experimental.pallas import tpu_sc as plsc

_EMB_DIM = 64
_EMB_SIZE = 1024
_N = 9216

_R = 1024
_NB = _N // _R

_NC = 2
_NS = 16
_NW = _NC * _NS
_BPW = _N // _NW
_CH = 96
_NCH = _BPW // _CH


def _argmin_body(flat_ref, w_ref, wsq_ref, idx_ref):
    flat = flat_ref[...]
    w = w_ref[...]
    s2 = lax.dot_general(-2.0 * flat, w, (((1,), (1,)), ((), ())),
                         preferred_element_type=jnp.float32)
    xsq = jnp.sum(flat * flat, axis=1, keepdims=True)
    dist = (wsq_ref[...] + s2) + xsq
    m = jnp.min(dist, axis=1, keepdims=True)
    cols = lax.broadcasted_iota(jnp.int32, dist.shape, 1)
    idx_ref[...] = jnp.min(jnp.where(dist == m, cols, _EMB_SIZE), axis=1,
                           keepdims=True)


def _argmin_indices(flat, W, wsq):
    return pl.pallas_call(
        _argmin_body,
        grid=(_NB,),
        in_specs=[
            pl.BlockSpec((_R, _EMB_DIM), lambda i: (i, 0)),
            pl.BlockSpec((_EMB_SIZE, _EMB_DIM), lambda i: (0, 0)),
            pl.BlockSpec((1, _EMB_SIZE), lambda i: (0, 0)),
        ],
        out_specs=pl.BlockSpec((_R, 1), lambda i: (i, 0)),
        out_shape=jax.ShapeDtypeStruct((_N, 1), jnp.int32),
    )(flat, W, wsq)


def _gather_body(table_hbm, idx_hbm, out_hbm, idx_v, rows_v, sem):
    wid = lax.axis_index("s") * _NC + lax.axis_index("c")
    pltpu.sync_copy(idx_hbm.at[wid], idx_v)
    copies = [
        pltpu.async_copy(table_hbm.at[idx_v.at[c]],
                         rows_v.at[pl.ds(c * _CH, _CH)], sem)
        for c in range(_NCH)
    ]
    for cp in copies:
        cp.wait()
    pltpu.sync_copy(rows_v, out_hbm.at[pl.ds(wid * _BPW, _BPW)])


def _gather_rows(Wpad, idx3):
    f = functools.partial(
        pl.kernel,
        out_type=jax.ShapeDtypeStruct((_N, 128), jnp.float32),
        mesh=plsc.VectorSubcoreMesh(core_axis_name="c", subcore_axis_name="s",
                                    num_cores=_NC, num_subcores=_NS),
        scratch_types=[
            pltpu.VMEM((_NCH, _CH), jnp.int32),
            pltpu.VMEM((_BPW, 128), jnp.float32),
            pltpu.SemaphoreType.DMA,
        ],
    )(_gather_body)
    return f(Wpad, idx3)


def kernel(x, W):
    B, T, D = x.shape
    flat = x.reshape(_N, D)
    wsq = jnp.sum(W * W, axis=1)[None, :]
    idx = _argmin_indices(flat, W, wsq)
    Wpad = jnp.pad(W, ((0, 0), (0, 128 - _EMB_DIM)))
    embed = _gather_rows(Wpad, idx.reshape(_NW, _NCH, _CH))
    embed = embed[:, :_EMB_DIM].reshape(B, T, D)
    return (embed, embed, idx.reshape(B, T))


# speedup vs baseline: 1.0343x; 1.0343x over previous
import functools

import jax
import jax.numpy as jnp
from jax import lax
from jax.experimental import pallas as pl
from jax.experimental.pallas import tpu as pltpu
from jax.experimental.pallas import tpu_sc as plsc

_EMB_DIM = 64
_EMB_SIZE = 1024
_N = 9216

_R = 1152
_NB = _N // _R
_BB = 2

_NC = 2
_NS = 16
_NW = _NC * _NS
_BPW = _N // _NW
_CH = 96
_NCH = _BPW // _CH


def _argmin_body(x_ref, w_ref, wsq_ref, xsq_ref, idx_ref):
    flat = x_ref[...].reshape(_R, _EMB_DIM)
    w2 = -2.0 * w_ref[...]
    s2t = lax.dot_general(w2, flat, (((1,), (1,)), ((), ())),
                          preferred_element_type=jnp.float32)
    dist = (wsq_ref[...] + s2t) + xsq_ref[...]
    m = jnp.min(dist, axis=0, keepdims=True)
    rows = lax.broadcasted_iota(jnp.int32, dist.shape, 0)
    idx = jnp.min(jnp.where(dist == m, rows, _EMB_SIZE), axis=0)
    idx_ref[...] = idx.reshape(1, 1, _R)


def _argmin_indices(x, W, wsq, xsq):
    return pl.pallas_call(
        _argmin_body,
        grid=(_NB,),
        in_specs=[
            pl.BlockSpec((_BB, 576, _EMB_DIM), lambda i: (i, 0, 0)),
            pl.BlockSpec((_EMB_SIZE, _EMB_DIM), lambda i: (0, 0)),
            pl.BlockSpec((_EMB_SIZE, 1), lambda i: (0, 0)),
            pl.BlockSpec((1, _R), lambda i: (0, i)),
        ],
        out_specs=pl.BlockSpec((1, 1, _R), lambda i: (i, 0, 0)),
        out_shape=jax.ShapeDtypeStruct((_NB, 1, _R), jnp.int32),
    )(x, W, wsq, xsq)


def _gather_body(table_hbm, idx_hbm, out_hbm, idx_v, rows_v, sem):
    wid = lax.axis_index("s") * _NC + lax.axis_index("c")
    pltpu.sync_copy(idx_hbm.at[wid], idx_v)
    copies = [
        pltpu.async_copy(table_hbm.at[idx_v.at[c]],
                         rows_v.at[pl.ds(c * _CH, _CH)], sem)
        for c in range(_NCH)
    ]
    for cp in copies:
        cp.wait()
    pltpu.sync_copy(rows_v, out_hbm.at[pl.ds(wid * _BPW, _BPW)])


def _gather_rows(Wpad, idx3):
    f = functools.partial(
        pl.kernel,
        out_type=jax.ShapeDtypeStruct((_N, 128), jnp.float32),
        mesh=plsc.VectorSubcoreMesh(core_axis_name="c", subcore_axis_name="s",
                                    num_cores=_NC, num_subcores=_NS),
        scratch_types=[
            pltpu.VMEM((_NCH, _CH), jnp.int32),
            pltpu.VMEM((_BPW, 128), jnp.float32),
            pltpu.SemaphoreType.DMA,
        ],
    )(_gather_body)
    return f(Wpad, idx3)


def kernel(x, W):
    B, T, D = x.shape
    wsq = jnp.sum(W * W, axis=1, keepdims=True)
    xsq = jnp.sum(x * x, axis=2).reshape(1, _N)
    idx = _argmin_indices(x, W, wsq, xsq)
    Wpad = jnp.pad(W, ((0, 0), (0, 128 - _EMB_DIM)))
    embed = _gather_rows(Wpad, idx.reshape(_NW, _NCH, _CH))
    embed = embed[:, :_EMB_DIM].reshape(B, T, D)
    return (embed, embed, idx.reshape(B, T))


# speedup vs baseline: 1.0513x; 1.0164x over previous
import functools

import jax
import jax.numpy as jnp
from jax import lax
from jax.experimental import pallas as pl
from jax.experimental.pallas import tpu as pltpu
from jax.experimental.pallas import tpu_sc as plsc

_EMB_DIM = 64
_EMB_SIZE = 1024
_N = 9216

_R = 576
_NB = _N // _R

_NC = 2
_NS = 16
_NW = _NC * _NS
_BPW = _N // _NW
_CH = 96
_NCH = _BPW // _CH


def _argmin_body(xt_ref, w_ref, wsq_ref, xsq_ref, idx_ref):
    flat_t = xt_ref[...].reshape(_EMB_DIM, _R)
    w2 = -2.0 * w_ref[...]
    s2t = lax.dot_general(w2, flat_t, (((1,), (0,)), ((), ())),
                          preferred_element_type=jnp.float32)
    dist = (wsq_ref[...] + s2t) + xsq_ref[...].reshape(1, _R)
    m = jnp.min(dist, axis=0, keepdims=True)
    rows = lax.broadcasted_iota(jnp.int32, dist.shape, 0)
    idx = jnp.min(jnp.where(dist == m, rows, _EMB_SIZE), axis=0)
    idx_ref[...] = idx.reshape(1, 1, _R)


def _argmin_indices(xt, W, wsq, xsq):
    return pl.pallas_call(
        _argmin_body,
        grid=(_NB,),
        in_specs=[
            pl.BlockSpec((1, _EMB_DIM, _R), lambda i: (i, 0, 0)),
            pl.BlockSpec((_EMB_SIZE, _EMB_DIM), lambda i: (0, 0)),
            pl.BlockSpec((_EMB_SIZE, 1), lambda i: (0, 0)),
            pl.BlockSpec((1, 1, _R), lambda i: (i, 0, 0)),
        ],
        out_specs=pl.BlockSpec((1, 1, _R), lambda i: (i, 0, 0)),
        out_shape=jax.ShapeDtypeStruct((_NB, 1, _R), jnp.int32),
    )(xt, W, wsq, xsq)


def _gather_body(table_hbm, idx_hbm, out_hbm, idx_v, rows_v, sem):
    wid = lax.axis_index("s") * _NC + lax.axis_index("c")
    pltpu.sync_copy(idx_hbm.at[wid], idx_v)
    copies = [
        pltpu.async_copy(table_hbm.at[idx_v.at[c]],
                         rows_v.at[pl.ds(c * _CH, _CH)], sem)
        for c in range(_NCH)
    ]
    for cp in copies:
        cp.wait()
    pltpu.sync_copy(rows_v, out_hbm.at[pl.ds(wid * _BPW, _BPW)])


def _gather_rows(Wpad, idx3):
    f = functools.partial(
        pl.kernel,
        out_type=jax.ShapeDtypeStruct((_N, 128), jnp.float32),
        mesh=plsc.VectorSubcoreMesh(core_axis_name="c", subcore_axis_name="s",
                                    num_cores=_NC, num_subcores=_NS),
        scratch_types=[
            pltpu.VMEM((_NCH, _CH), jnp.int32),
            pltpu.VMEM((_BPW, 128), jnp.float32),
            pltpu.SemaphoreType.DMA,
        ],
    )(_gather_body)
    return f(Wpad, idx3)


def kernel(x, W):
    B, T, D = x.shape
    wsq = jnp.sum(W * W, axis=1, keepdims=True)
    xsq = jnp.sum(x * x, axis=2).reshape(_NB, 1, _R)
    xt = x.transpose(0, 2, 1)
    idx = _argmin_indices(xt, W, wsq, xsq)
    Wpad = jnp.pad(W, ((0, 0), (0, 128 - _EMB_DIM)))
    embed = _gather_rows(Wpad, idx.reshape(_NW, _NCH, _CH))
    embed = embed[:, :_EMB_DIM].reshape(B, T, D)
    return (embed, embed, idx.reshape(B, T))


# speedup vs baseline: 1.1044x; 1.0506x over previous
import functools

import jax
import jax.numpy as jnp
from jax import lax
from jax.experimental import pallas as pl
from jax.experimental.pallas import tpu as pltpu
from jax.experimental.pallas import tpu_sc as plsc

_EMB_DIM = 64
_EMB_SIZE = 1024
_N = 9216

_R = 576
_NB = _N // _R
_BB = 4

_NC = 2
_NS = 16
_NW = _NC * _NS
_BPW = _N // _NW
_CH = 96
_NCH = _BPW // _CH


def _argmin_body(xt_ref, w_ref, wsq_ref, xsq_ref, idx_ref):
    w2 = -2.0 * w_ref[...]
    for b in range(_BB):
        flat_t = xt_ref[b]
        s2t = lax.dot_general(w2, flat_t, (((1,), (0,)), ((), ())),
                              preferred_element_type=jnp.float32)
        dist = (wsq_ref[...] + s2t) + xsq_ref[b]
        m = jnp.min(dist, axis=0, keepdims=True)
        rows = lax.broadcasted_iota(jnp.int32, dist.shape, 0)
        idx = jnp.min(jnp.where(dist == m, rows, _EMB_SIZE), axis=0)
        idx_ref[0, b, :] = idx


def _argmin_indices(xt, W, wsq, xsq):
    return pl.pallas_call(
        _argmin_body,
        grid=(_NB // _BB,),
        in_specs=[
            pl.BlockSpec((_BB, _EMB_DIM, _R), lambda i: (i, 0, 0)),
            pl.BlockSpec((_EMB_SIZE, _EMB_DIM), lambda i: (0, 0)),
            pl.BlockSpec((_EMB_SIZE, 1), lambda i: (0, 0)),
            pl.BlockSpec((_BB, 1, _R), lambda i: (i, 0, 0)),
        ],
        out_specs=pl.BlockSpec((1, _BB, _R), lambda i: (i, 0, 0)),
        out_shape=jax.ShapeDtypeStruct((_NB // _BB, _BB, _R), jnp.int32),
    )(xt, W, wsq, xsq)


def _gather_body(table_hbm, idx_hbm, out_hbm, idx_v, rows_v, sem):
    wid = lax.axis_index("s") * _NC + lax.axis_index("c")
    pltpu.sync_copy(idx_hbm.at[wid], idx_v)
    copies = [
        pltpu.async_copy(table_hbm.at[idx_v.at[c]],
                         rows_v.at[pl.ds(c * _CH, _CH)], sem)
        for c in range(_NCH)
    ]
    for cp in copies:
        cp.wait()
    pltpu.sync_copy(rows_v, out_hbm.at[pl.ds(wid * _BPW, _BPW)])


def _gather_rows(Wpad, idx3):
    f = functools.partial(
        pl.kernel,
        out_type=jax.ShapeDtypeStruct((_N, 128), jnp.float32),
        mesh=plsc.VectorSubcoreMesh(core_axis_name="c", subcore_axis_name="s",
                                    num_cores=_NC, num_subcores=_NS),
        scratch_types=[
            pltpu.VMEM((_NCH, _CH), jnp.int32),
            pltpu.VMEM((_BPW, 128), jnp.float32),
            pltpu.SemaphoreType.DMA,
        ],
    )(_gather_body)
    return f(Wpad, idx3)


def kernel(x, W):
    B, T, D = x.shape
    wsq = jnp.sum(W * W, axis=1, keepdims=True)
    xsq = jnp.sum(x * x, axis=2).reshape(_NB, 1, _R)
    xt = x.transpose(0, 2, 1)
    idx = _argmin_indices(xt, W, wsq, xsq)
    Wpad = jnp.pad(W, ((0, 0), (0, 128 - _EMB_DIM)))
    embed = _gather_rows(Wpad, idx.reshape(_NW, _NCH, _CH))
    embed = embed[:, :_EMB_DIM].reshape(B, T, D)
    return (embed, embed, idx.reshape(B, T))
